# baseline (device time: 102952 ns/iter reference)
import jax
import jax.numpy as jnp
from jax import lax
from jax.experimental import pallas as pl
from jax.experimental.pallas import tpu as pltpu

N_DEV = 16
NSUB = 4


def kernel(x):
    m, n = x.shape
    mc = m // (2 * N_DEV)
    sub = mc // NSUB
    xr = x.reshape(2 * N_DEV, mc, n)

    def body(x_ref, out_ref, sbuf_a, sbuf_b, rbuf_a, rbuf_b,
             ssem_a, ssem_b, rsem_a, rsem_b,
             ag_ssem_a, ag_ssem_b, ag_rsem_a, ag_rsem_b):
        my = lax.axis_index("i")
        q = lax.rem(my, 4)
        z = my // 4
        rp = jnp.where(
            q == 0, z,
            jnp.where(q == 1, 7 - z, jnp.where(q == 2, 8 + z, 15 - z)))
        right = jnp.where(
            (q == 0) | (q == 2),
            jnp.where(z < 3, my + 4, my + 1),
            jnp.where(z > 0, my - 4, jnp.where(q == 1, my + 1, my - 3)))
        left = jnp.where(
            q == 0,
            jnp.where(z > 0, my - 4, my + 3),
            jnp.where((q == 1) | (q == 3),
                      jnp.where(z < 3, my + 4, my - 1),
                      jnp.where(z > 0, my - 4, my - 1)))

        bar = pltpu.get_barrier_semaphore()
        for nbr in (left, right):
            pl.semaphore_signal(
                bar, inc=1, device_id=(nbr,),
                device_id_type=pl.DeviceIdType.MESH,
            )
        pl.semaphore_wait(bar, 2)

        rows = [pl.ds(k * sub, sub) for k in range(NSUB)]
        rs_descs = {"a": [], "b": []}
        ag_descs = {"a": [], "b": []}

        cast = lambda v: v.astype(jnp.bfloat16)

        def rs_desc(d, s, k, src):
            rbuf, rsem, ssem, peer = (
                (rbuf_a, rsem_a, ssem_a, right) if d == "a"
                else (rbuf_b, rsem_b, ssem_b, left)
            )
            return pltpu.make_async_remote_copy(
                src_ref=src,
                dst_ref=rbuf.at[s, rows[k]],
                send_sem=ssem.at[s, k],
                recv_sem=rsem.at[s, k],
                device_id=(peer,),
                device_id_type=pl.DeviceIdType.MESH,
            )

        def ag_desc(d, t, k, c):
            ssem, rsem, peer = (
                (ag_ssem_a, ag_rsem_a, right) if d == "a"
                else (ag_ssem_b, ag_rsem_b, left)
            )
            return pltpu.make_async_remote_copy(
                src_ref=out_ref.at[c, rows[k]],
                dst_ref=out_ref.at[c, rows[k]],
                send_sem=ssem.at[t, k],
                recv_sem=rsem.at[t, k],
                device_id=(peer,),
                device_id_type=pl.DeviceIdType.MESH,
            )

        for s in range(N_DEV - 1):
            ca = lax.rem(rp - s + 2 * N_DEV, N_DEV)
            cb = N_DEV + lax.rem(rp + s, N_DEV)
            hop_a, hop_b = [], []
            for k in range(NSUB):
                for d, c, rbuf, sbuf, hop in (
                    ("a", ca, rbuf_a, sbuf_a, hop_a),
                    ("b", cb, rbuf_b, sbuf_b, hop_b),
                ):
                    if s == 0:
                        sbuf[rows[k]] = cast(x_ref[c, rows[k]])
                        src = sbuf.at[rows[k]]
                    else:
                        rs_descs[d][s - 1][k].wait_recv()
                        rbuf[s - 1, rows[k]] = (
                            rbuf[s - 1, rows[k]] + cast(x_ref[c, rows[k]])
                        )
                        src = rbuf.at[s - 1, rows[k]]
                    desc = rs_desc(d, s, k, src)
                    desc.start()
                    hop.append(desc)
            rs_descs["a"].append(hop_a)
            rs_descs["b"].append(hop_b)

        c_mine_a = lax.rem(rp + 1, N_DEV)
        c_mine_b = N_DEV + lax.rem(rp + N_DEV - 1, N_DEV)

        for t in range(N_DEV - 1):
            ca = lax.rem(rp + 1 - t + 2 * N_DEV, N_DEV)
            cb = N_DEV + lax.rem(rp - 1 + t + 2 * N_DEV, N_DEV)
            hop_a, hop_b = [], []
            for k in range(NSUB):
                for d, c, c_mine, rbuf, hop in (
                    ("a", ca, c_mine_a, rbuf_a, hop_a),
                    ("b", cb, c_mine_b, rbuf_b, hop_b),
                ):
                    if t == 0:
                        rs_descs[d][N_DEV - 2][k].wait_recv()
                        out_ref[c_mine, rows[k]] = (
                            rbuf[N_DEV - 2, rows[k]]
                            + cast(x_ref[c_mine, rows[k]])
                        )
                    else:
                        ag_descs[d][t - 1][k].wait_recv()
                    desc = ag_desc(d, t, k, c)
                    desc.start()
                    hop.append(desc)
            ag_descs["a"].append(hop_a)
            ag_descs["b"].append(hop_b)

        for d in ("a", "b"):
            for k in range(NSUB):
                ag_descs[d][N_DEV - 2][k].wait_recv()
        for d in ("a", "b"):
            for s in range(N_DEV - 1):
                for k in range(NSUB):
                    rs_descs[d][s][k].wait_send()
                    ag_descs[d][s][k].wait_send()

    out = pl.pallas_call(
        body,
        out_shape=jax.ShapeDtypeStruct((2 * N_DEV, mc, n), jnp.bfloat16),
        in_specs=[pl.BlockSpec(memory_space=pltpu.VMEM)],
        out_specs=pl.BlockSpec(memory_space=pltpu.VMEM),
        scratch_shapes=[
            pltpu.VMEM((mc, n), jnp.bfloat16),
            pltpu.VMEM((mc, n), jnp.bfloat16),
            pltpu.VMEM((N_DEV - 1, mc, n), jnp.bfloat16),
            pltpu.VMEM((N_DEV - 1, mc, n), jnp.bfloat16),
            pltpu.SemaphoreType.DMA((N_DEV - 1, NSUB)),
            pltpu.SemaphoreType.DMA((N_DEV - 1, NSUB)),
            pltpu.SemaphoreType.DMA((N_DEV - 1, NSUB)),
            pltpu.SemaphoreType.DMA((N_DEV - 1, NSUB)),
            pltpu.SemaphoreType.DMA((N_DEV - 1, NSUB)),
            pltpu.SemaphoreType.DMA((N_DEV - 1, NSUB)),
            pltpu.SemaphoreType.DMA((N_DEV - 1, NSUB)),
            pltpu.SemaphoreType.DMA((N_DEV - 1, NSUB)),
        ],
        compiler_params=pltpu.CompilerParams(collective_id=0),
    )(xr)
    return out.reshape(m, n)
